# baseline (speedup 1.0000x reference)
import functools

import jax
import jax.numpy as jnp
from jax import lax
from jax.experimental import pallas as pl
from jax.experimental.pallas import tpu as pltpu
from jax.experimental.pallas import tpu_sc as plsc

NC, NS = 2, 16
NW = NC * NS
HALF = 100
NRING = 4


def _sc_gather(x3, table, R, W, D):
    r_per_w = R // NW
    mesh = plsc.VectorSubcoreMesh(core_axis_name="c", subcore_axis_name="s")

    @functools.partial(
        pl.kernel,
        out_type=jax.ShapeDtypeStruct((R, W, D), jnp.float32),
        mesh=mesh,
        scratch_types=[
            pltpu.VMEM((NRING, 2, HALF), jnp.int32),
            pltpu.VMEM((NRING, W, D), jnp.float32),
            [pltpu.SemaphoreType.DMA] * NRING,
            [pltpu.SemaphoreType.DMA] * NRING,
            [pltpu.SemaphoreType.DMA] * NRING,
        ],
        compiler_params=pltpu.CompilerParams(use_tc_tiling_on_sc=False),
    )
    def k(x_hbm, table_hbm, out_hbm, idx_v, rows_v, sem_idx, sem_g, sem_out):
        wid = lax.axis_index("s") * NC + lax.axis_index("c")
        r0 = wid * r_per_w

        def start_idx(r, slot):
            @pl.when(r < r_per_w)
            def _():
                pltpu.async_copy(x_hbm.at[r0 + r], idx_v.at[slot], sem_idx[slot])

        def wait_idx(slot):
            pltpu.make_async_copy(
                x_hbm.at[r0], idx_v.at[slot], sem_idx[slot]
            ).wait()

        def wait_out(slot):
            pltpu.make_async_copy(
                rows_v.at[slot], out_hbm.at[r0], sem_out[slot]
            ).wait()

        def row_step(r, slot, first_round):
            if not first_round:
                wait_out(slot)
            wait_idx(slot)
            copies = [
                pltpu.async_copy(
                    table_hbm.at[idx_v.at[slot, h]],
                    rows_v.at[slot, pl.ds(h * HALF, HALF)],
                    sem_g[slot],
                )
                for h in range(2)
            ]
            for c in copies:
                c.wait()
            pltpu.async_copy(rows_v.at[slot], out_hbm.at[r0 + r], sem_out[slot])
            start_idx(r + NRING, slot)

        for slot in range(NRING):
            start_idx(jnp.int32(slot), slot)
        for slot in range(NRING):
            row_step(jnp.int32(slot), slot, first_round=True)

        def body(i, carry):
            g0 = i * NRING
            for slot in range(NRING):
                row_step(g0 + slot, slot, first_round=False)
            return carry

        lax.fori_loop(1, r_per_w // NRING, body, 0)
        for slot in range(NRING):
            wait_out(slot)

    return k(x3, table)


def kernel(x, table):
    R, W = x.shape
    D = table.shape[1]
    x3 = x.reshape(R, 2, HALF).astype(jnp.int32)
    return _sc_gather(x3, table, R, W, D)


# speedup vs baseline: 4.9147x; 4.9147x over previous
import functools

import jax
import jax.numpy as jnp
from jax import lax
from jax.experimental import pallas as pl
from jax.experimental.pallas import tpu as pltpu
from jax.experimental.pallas import tpu_sc as plsc

NC, NS = 2, 16
NW = NC * NS
L = 16
NIDX = 4
NBUF = 4


STRIDE = 33


def _sc_lookup(xq, table_flat, V, D, TJ, TI, SJ, LI):
    ti_per_w = TI // NW
    nj = TJ * SJ
    tk = D // 8
    mesh = plsc.VectorSubcoreMesh(core_axis_name="c", subcore_axis_name="s")

    @functools.partial(
        pl.kernel,
        out_type=jax.ShapeDtypeStruct((nj, tk, TI, 8, LI), jnp.float32),
        mesh=mesh,
        scratch_types=[
            pltpu.VMEM((V * STRIDE,), jnp.float32),
            pltpu.VMEM((NIDX, LI), jnp.int32),
            pltpu.VMEM((NBUF, tk, 8, LI), jnp.float32),
            [pltpu.SemaphoreType.DMA] * NIDX,
            [pltpu.SemaphoreType.DMA] * NBUF,
            pltpu.SemaphoreType.DMA,
        ],
        compiler_params=pltpu.CompilerParams(
            use_tc_tiling_on_sc=False, needs_layout_passes=False
        ),
    )
    def k(x_hbm, tbl_hbm, out_hbm, tbl_v, idx_v, buf_v, sem_idx, sem_out, sem_t):
        wid = lax.axis_index("s") * NC + lax.axis_index("c")
        ti0 = wid * ti_per_w
        total = ti_per_w * nj

        pltpu.async_copy(tbl_hbm, tbl_v, sem_t).wait()

        def start_idx(n, slot):
            @pl.when(n < total)
            def _():
                ti = ti0 + n // nj
                j = n % nj
                pltpu.async_copy(
                    x_hbm.at[j // SJ, ti, j % SJ], idx_v.at[slot], sem_idx[slot]
                )

        def wait_idx(slot):
            pltpu.make_async_copy(
                x_hbm.at[0, 0, 0], idx_v.at[slot], sem_idx[slot]
            ).wait()

        def wait_out(slot):
            pltpu.make_async_copy(
                buf_v.at[slot], out_hbm.at[0, :, 0], sem_out[slot]
            ).wait()

        def step(n, islot, bslot, first_round):
            if not first_round:
                wait_out(bslot)
            wait_idx(islot)
            addrs = [
                idx_v[islot, pl.ds(g * L, L)] * STRIDE
                for g in range(LI // L)
            ]
            for kk in range(D):
                vals = [
                    plsc.load_gather(tbl_v, [addrs[g] + kk])
                    for g in range(LI // L)
                ]
                for g in range(LI // L):
                    buf_v[bslot, kk // 8, kk % 8, pl.ds(g * L, L)] = vals[g]
            ti = ti0 + n // nj
            j = n % nj
            pltpu.async_copy(
                buf_v.at[bslot], out_hbm.at[j, :, ti], sem_out[bslot]
            )
            start_idx(n + NIDX, islot)

        for s in range(NIDX):
            start_idx(jnp.int32(s), s)
        for s in range(NIDX):
            step(jnp.int32(s), s, s % NBUF, first_round=(s < NBUF))

        def body(i, carry):
            n0 = i * NIDX
            for s in range(NIDX):
                step(n0 + s, s, s % NBUF, first_round=False)
            return carry

        lax.fori_loop(1, total // NIDX, body, 0)
        for s in range(NBUF):
            wait_out(s)

    return k(xq, table_flat)


def kernel(x, table):
    R, W = x.shape
    V, D = table.shape
    TI, LI = R // 128, 128
    TJ, SJ = W // 8, 8
    xq = x.astype(jnp.int32).reshape(TI, LI, TJ, SJ).transpose(2, 0, 3, 1)
    tbl_pad = jnp.pad(table, ((0, 0), (0, STRIDE - D))).reshape(V * STRIDE)
    p = _sc_lookup(xq, tbl_pad, V, D, TJ, TI, SJ, LI)
    return p.transpose(2, 4, 0, 1, 3).reshape(R, W, D)
